# baseline (speedup 1.0000x reference)
import jax
import jax.numpy as jnp
from jax.experimental import pallas as pl
from jax.experimental.pallas import tpu as pltpu

N = 4096
BN = 256
NB = N // BN
B = 4
D_IN = 16
D_H = 32
D_CAT = D_IN + D_H
DXB = D_CAT * B
DH4 = D_H * B
DRU = 2 * D_H * B
M = 3
PREC = jax.lax.Precision.DEFAULT


def _dcgru_body(S_ref, x0_ref, Wru_ref, bru_ref, Wc_ref, bc_ref,
                out_ref, x1_s, x0p_s, x1p_s, u_s):
    p = pl.program_id(0)
    i = pl.program_id(1)
    rows = pl.ds(i * BN, BN)
    Sb = S_ref[...]

    def mm(a, b):
        return jax.lax.dot(a, b, precision=PREC,
                           preferred_element_type=jnp.float32)

    @pl.when(p == 0)
    def _():
        x1_s[rows, :] = mm(Sb, x0_ref[...])

    @pl.when(p == 1)
    def _():
        x0b = x0_ref[rows, :]
        x1b = x1_s[rows, :]
        x2b = 2.0 * mm(Sb, x1_s[...]) - x0b
        h = (mm(x0b, Wru_ref[0]) + mm(x1b, Wru_ref[1]) + mm(x2b, Wru_ref[2])
             + bru_ref[...])
        ru = jax.nn.sigmoid(h)
        r = ru[:, :DH4]
        u_s[rows, :] = ru[:, DH4:]
        x0p_s[rows, :] = jnp.concatenate(
            [x0b[:, :D_IN * B], r * x0b[:, D_IN * B:]], axis=1)

    @pl.when(p == 2)
    def _():
        x1p_s[rows, :] = mm(Sb, x0p_s[...])

    @pl.when(p == 3)
    def _():
        x0pb = x0p_s[rows, :]
        x1pb = x1p_s[rows, :]
        x2pb = 2.0 * mm(Sb, x1p_s[...]) - x0pb
        hc = (mm(x0pb, Wc_ref[0]) + mm(x1pb, Wc_ref[1]) + mm(x2pb, Wc_ref[2])
              + bc_ref[...])
        c = jnp.tanh(hc)
        u = u_s[rows, :]
        st = x0_ref[rows, D_IN * B:]
        out_ref[...] = u * st + (1.0 - u) * c


def _expand_weight(W, d_out):
    Wr = W.reshape(d_out, D_CAT, M)
    eye = jnp.eye(B, dtype=W.dtype)
    T = (Wr.transpose(2, 1, 0)[:, :, None, :, None]
         * eye[None, None, :, None, :])
    return T.reshape(M, D_CAT * B, d_out * B)


def kernel(inputs, supports, states, W_ru, b_ru, W_c, b_c):
    n = supports.shape[0]
    x = jnp.concatenate([inputs, states], axis=-1)
    x0 = x.transpose(1, 2, 0).reshape(n, DXB)
    Wru_cat = _expand_weight(W_ru, 2 * D_H)
    Wc_cat = _expand_weight(W_c, D_H)
    bru_f = jnp.repeat(b_ru, B).reshape(1, DRU)
    bc_f = jnp.repeat(b_c, B).reshape(1, DH4)

    out_flat = pl.pallas_call(
        _dcgru_body,
        grid=(4, NB),
        in_specs=[
            pl.BlockSpec((BN, N), lambda p, i: (i, 0)),
            pl.BlockSpec((N, DXB), lambda p, i: (0, 0)),
            pl.BlockSpec((M, DXB, DRU), lambda p, i: (0, 0, 0)),
            pl.BlockSpec((1, DRU), lambda p, i: (0, 0)),
            pl.BlockSpec((M, DXB, DH4), lambda p, i: (0, 0, 0)),
            pl.BlockSpec((1, DH4), lambda p, i: (0, 0)),
        ],
        out_specs=pl.BlockSpec((BN, DH4), lambda p, i: (i, 0)),
        out_shape=jax.ShapeDtypeStruct((N, DH4), jnp.float32),
        scratch_shapes=[
            pltpu.VMEM((N, DXB), jnp.float32),
            pltpu.VMEM((N, DXB), jnp.float32),
            pltpu.VMEM((N, DXB), jnp.float32),
            pltpu.VMEM((N, DH4), jnp.float32),
        ],
        compiler_params=pltpu.CompilerParams(
            dimension_semantics=("arbitrary", "arbitrary")),
    )(supports, x0, Wru_cat, bru_f, Wc_cat, bc_f)

    out = out_flat.reshape(n, D_H, B).transpose(2, 0, 1)
    return (out, out)


# speedup vs baseline: 1.1817x; 1.1817x over previous
import jax
import jax.numpy as jnp
from jax.experimental import pallas as pl
from jax.experimental.pallas import tpu as pltpu

N = 4096
BNA = 512
NBA = N // BNA
BNB = 1024
NBB = N // BNB
B = 4
D_IN = 16
D_H = 32
D_CAT = D_IN + D_H
DXB = D_CAT * B
DH4 = D_H * B
DRU = 2 * D_H * B
M = 3
F32 = jnp.float32
BF16 = jnp.bfloat16


def _mm(a, b):
    return jax.lax.dot(a, b, preferred_element_type=F32)


def _cast_body(S_ref, x016_ref, S16_ref, x116_ref):
    Sb16 = S_ref[...].astype(BF16)
    S16_ref[...] = Sb16
    x116_ref[...] = _mm(Sb16, x016_ref[...]).astype(BF16)


def _gru_body(S16_ref, x0f_ref, x016_ref, x116_ref, Wru_ref, bru_ref,
              Wc_ref, bc_ref, out_ref, x0pf_s, x0p16_s, x1p16_s, u_s):
    p = pl.program_id(0)
    i = pl.program_id(1)
    rows = pl.ds(i * BNB, BNB)
    Sb = S16_ref[...]

    @pl.when(p == 0)
    def _():
        x0b = x0f_ref[rows, :]
        x2b = 2.0 * _mm(Sb, x116_ref[...]) - x0b
        h = (_mm(x016_ref[rows, :], Wru_ref[0])
             + _mm(x116_ref[rows, :], Wru_ref[1])
             + _mm(x2b.astype(BF16), Wru_ref[2])
             + bru_ref[...])
        ru = jax.nn.sigmoid(h)
        r = ru[:, :DH4]
        u_s[rows, :] = ru[:, DH4:]
        x0pb = jnp.concatenate(
            [x0b[:, :D_IN * B], r * x0b[:, D_IN * B:]], axis=1)
        x0pf_s[rows, :] = x0pb
        x0p16_s[rows, :] = x0pb.astype(BF16)

    @pl.when(p == 1)
    def _():
        x1p16_s[rows, :] = _mm(Sb, x0p16_s[...]).astype(BF16)

    @pl.when(p == 2)
    def _():
        x0pb = x0pf_s[rows, :]
        x2pb = 2.0 * _mm(Sb, x1p16_s[...]) - x0pb
        hc = (_mm(x0p16_s[rows, :], Wc_ref[0])
              + _mm(x1p16_s[rows, :], Wc_ref[1])
              + _mm(x2pb.astype(BF16), Wc_ref[2])
              + bc_ref[...])
        c = jnp.tanh(hc)
        u = u_s[rows, :]
        st = x0f_ref[rows, D_IN * B:]
        out_ref[...] = u * st + (1.0 - u) * c


def _expand_weight(W, d_out):
    Wr = W.reshape(d_out, D_CAT, M)
    eye = jnp.eye(B, dtype=W.dtype)
    T = (Wr.transpose(2, 1, 0)[:, :, None, :, None]
         * eye[None, None, :, None, :])
    return T.reshape(M, D_CAT * B, d_out * B).astype(BF16)


def kernel(inputs, supports, states, W_ru, b_ru, W_c, b_c):
    n = supports.shape[0]
    x = jnp.concatenate([inputs, states], axis=-1)
    x0 = x.transpose(1, 2, 0).reshape(n, DXB)
    x016 = x0.astype(BF16)
    Wru_cat = _expand_weight(W_ru, 2 * D_H)
    Wc_cat = _expand_weight(W_c, D_H)
    bru_f = jnp.repeat(b_ru, B).reshape(1, DRU)
    bc_f = jnp.repeat(b_c, B).reshape(1, DH4)

    S16, x116 = pl.pallas_call(
        _cast_body,
        grid=(NBA,),
        in_specs=[
            pl.BlockSpec((BNA, N), lambda i: (i, 0)),
            pl.BlockSpec((N, DXB), lambda i: (0, 0)),
        ],
        out_specs=[
            pl.BlockSpec((BNA, N), lambda i: (i, 0)),
            pl.BlockSpec((BNA, DXB), lambda i: (i, 0)),
        ],
        out_shape=[
            jax.ShapeDtypeStruct((n, n), BF16),
            jax.ShapeDtypeStruct((n, DXB), BF16),
        ],
        compiler_params=pltpu.CompilerParams(
            dimension_semantics=("arbitrary",)),
    )(supports, x016)

    out_flat = pl.pallas_call(
        _gru_body,
        grid=(3, NBB),
        in_specs=[
            pl.BlockSpec((BNB, N), lambda p, i: (i, 0)),
            pl.BlockSpec((N, DXB), lambda p, i: (0, 0)),
            pl.BlockSpec((N, DXB), lambda p, i: (0, 0)),
            pl.BlockSpec((N, DXB), lambda p, i: (0, 0)),
            pl.BlockSpec((M, DXB, DRU), lambda p, i: (0, 0, 0)),
            pl.BlockSpec((1, DRU), lambda p, i: (0, 0)),
            pl.BlockSpec((M, DXB, DH4), lambda p, i: (0, 0, 0)),
            pl.BlockSpec((1, DH4), lambda p, i: (0, 0)),
        ],
        out_specs=pl.BlockSpec((BNB, DH4), lambda p, i: (i, 0)),
        out_shape=jax.ShapeDtypeStruct((n, DH4), F32),
        scratch_shapes=[
            pltpu.VMEM((N, DXB), F32),
            pltpu.VMEM((N, DXB), BF16),
            pltpu.VMEM((N, DXB), BF16),
            pltpu.VMEM((N, DH4), F32),
        ],
        compiler_params=pltpu.CompilerParams(
            dimension_semantics=("arbitrary", "arbitrary")),
    )(S16, x0, x016, x116, Wru_cat, bru_f, Wc_cat, bc_f)

    out = out_flat.reshape(n, D_H, B).transpose(2, 0, 1)
    return (out, out)


# speedup vs baseline: 1.2036x; 1.0185x over previous
import jax
import jax.numpy as jnp
from jax.experimental import pallas as pl
from jax.experimental.pallas import tpu as pltpu

N = 4096
BN = 256
NB = N // BN
B = 4
D_IN = 16
D_H = 32
D_CAT = D_IN + D_H
DXB = D_CAT * B
DH4 = D_H * B
DRU = 2 * D_H * B
M = 3
F32 = jnp.float32
BF16 = jnp.bfloat16


def _mm(a, b):
    return jax.lax.dot(a, b, preferred_element_type=F32)


def _gru_body(S_ref, x0f_ref, x016_ref, Wru_ref, bru_ref, Wc_ref, bc_ref,
              out_ref, S16_s, x116_s, x0p16_s, x1p16_s, u_s):
    p = pl.program_id(0)
    i = pl.program_id(1)
    rows = pl.ds(i * BN, BN)

    @pl.when(p == 0)
    def _():
        Sb16 = S_ref[...].astype(BF16)
        S16_s[rows, :] = Sb16
        x116_s[rows, :] = _mm(Sb16, x016_ref[...]).astype(BF16)

    @pl.when(p == 1)
    def _():
        Sb = S16_s[rows, :]
        x0b = x0f_ref[rows, :]
        x2b = 2.0 * _mm(Sb, x116_s[...]) - x0b
        h = (_mm(x016_ref[rows, :], Wru_ref[0])
             + _mm(x116_s[rows, :], Wru_ref[1])
             + _mm(x2b.astype(BF16), Wru_ref[2])
             + bru_ref[...])
        ru = jax.nn.sigmoid(h)
        r = ru[:, :DH4]
        u_s[rows, :] = ru[:, DH4:]
        x0pb = jnp.concatenate(
            [x0b[:, :D_IN * B], r * x0b[:, D_IN * B:]], axis=1)
        x0p16_s[rows, :] = x0pb.astype(BF16)

    @pl.when(p == 2)
    def _():
        x1p16_s[rows, :] = _mm(S16_s[rows, :], x0p16_s[...]).astype(BF16)

    @pl.when(p == 3)
    def _():
        x0pb = x0p16_s[rows, :].astype(F32)
        x2pb = 2.0 * _mm(S16_s[rows, :], x1p16_s[...]) - x0pb
        hc = (_mm(x0p16_s[rows, :], Wc_ref[0])
              + _mm(x1p16_s[rows, :], Wc_ref[1])
              + _mm(x2pb.astype(BF16), Wc_ref[2])
              + bc_ref[...])
        c = jnp.tanh(hc)
        u = u_s[rows, :]
        st = x0f_ref[rows, D_IN * B:]
        out_ref[...] = u * st + (1.0 - u) * c


def _expand_weight(W, d_out):
    Wr = W.reshape(d_out, D_CAT, M)
    eye = jnp.eye(B, dtype=W.dtype)
    T = (Wr.transpose(2, 1, 0)[:, :, None, :, None]
         * eye[None, None, :, None, :])
    return T.reshape(M, D_CAT * B, d_out * B).astype(BF16)


def kernel(inputs, supports, states, W_ru, b_ru, W_c, b_c):
    n = supports.shape[0]
    x = jnp.concatenate([inputs, states], axis=-1)
    x0 = x.transpose(1, 2, 0).reshape(n, DXB)
    x016 = x0.astype(BF16)
    Wru_cat = _expand_weight(W_ru, 2 * D_H)
    Wc_cat = _expand_weight(W_c, D_H)
    bru_f = jnp.repeat(b_ru, B).reshape(1, DRU)
    bc_f = jnp.repeat(b_c, B).reshape(1, DH4)

    out_flat = pl.pallas_call(
        _gru_body,
        grid=(4, NB),
        in_specs=[
            pl.BlockSpec((BN, N), lambda p, i: ((p == 0) * i, 0)),
            pl.BlockSpec((N, DXB), lambda p, i: (0, 0)),
            pl.BlockSpec((N, DXB), lambda p, i: (0, 0)),
            pl.BlockSpec((M, DXB, DRU), lambda p, i: (0, 0, 0)),
            pl.BlockSpec((1, DRU), lambda p, i: (0, 0)),
            pl.BlockSpec((M, DXB, DH4), lambda p, i: (0, 0, 0)),
            pl.BlockSpec((1, DH4), lambda p, i: (0, 0)),
        ],
        out_specs=pl.BlockSpec((BN, DH4), lambda p, i: (i, 0)),
        out_shape=jax.ShapeDtypeStruct((n, DH4), F32),
        scratch_shapes=[
            pltpu.VMEM((N, N), BF16),
            pltpu.VMEM((N, DXB), BF16),
            pltpu.VMEM((N, DXB), BF16),
            pltpu.VMEM((N, DXB), BF16),
            pltpu.VMEM((N, DH4), F32),
        ],
        compiler_params=pltpu.CompilerParams(
            dimension_semantics=("arbitrary", "arbitrary")),
    )(supports, x0, x016, Wru_cat, bru_f, Wc_cat, bc_f)

    out = out_flat.reshape(n, D_H, B).transpose(2, 0, 1)
    return (out, out)


# speedup vs baseline: 1.3308x; 1.1057x over previous
import jax
import jax.numpy as jnp
from jax.experimental import pallas as pl
from jax.experimental.pallas import tpu as pltpu

N = 4096
BN = 128
NB = N // BN
B = 4
D_IN = 16
D_H = 32
D_CAT = D_IN + D_H
DXB = D_CAT * B
DH4 = D_H * B
M = 3
F32 = jnp.float32
BF16 = jnp.bfloat16


def _mm(a, b):
    return jax.lax.dot(a, b, preferred_element_type=F32)


def _x0_rows(in2, st2):
    return jnp.concatenate(
        [jnp.concatenate([in2[:, b * D_IN:(b + 1) * D_IN],
                          st2[:, b * D_H:(b + 1) * D_H]], axis=1)
         for b in range(B)], axis=1)


def _gru_body(S_ref, in2_ref, st2_ref, Wru_ref, bru_ref, Wc_ref, bc_ref,
              out_ref, S16_s, x016_s, x116_s, x0p16_s, x1p16_s, u16_s):
    p = pl.program_id(0)
    i = pl.program_id(1)
    rows = pl.ds(i * BN, BN)

    @pl.when((p == 0) & (i == 0))
    def _():
        x016_s[...] = _x0_rows(in2_ref[...], st2_ref[...]).astype(BF16)

    @pl.when(p == 0)
    def _():
        Sb16 = S_ref[...].astype(BF16)
        S16_s[rows, :] = Sb16
        x116_s[rows, :] = _mm(Sb16, x016_s[...]).astype(BF16)

    @pl.when(p == 1)
    def _():
        x0b = _x0_rows(in2_ref[rows, :], st2_ref[rows, :])
        x2b = 2.0 * _mm(S16_s[rows, :], x116_s[...]) - x0b
        h = (_mm(x016_s[rows, :], Wru_ref[0])
             + _mm(x116_s[rows, :], Wru_ref[1])
             + _mm(x2b.astype(BF16), Wru_ref[2])
             + jnp.tile(bru_ref[...], (1, B)))
        ru = jax.nn.sigmoid(h)
        r = jnp.concatenate(
            [ru[:, b * 2 * D_H: b * 2 * D_H + D_H] for b in range(B)],
            axis=1)
        u16_s[rows, :] = jnp.concatenate(
            [ru[:, b * 2 * D_H + D_H: (b + 1) * 2 * D_H] for b in range(B)],
            axis=1).astype(BF16)
        rst = r * st2_ref[rows, :]
        x0p = jnp.concatenate(
            [jnp.concatenate([in2_ref[rows, b * D_IN:(b + 1) * D_IN],
                              rst[:, b * D_H:(b + 1) * D_H]], axis=1)
             for b in range(B)], axis=1)
        x0p16_s[rows, :] = x0p.astype(BF16)

    @pl.when(p == 2)
    def _():
        x1p16_s[rows, :] = _mm(S16_s[rows, :], x0p16_s[...]).astype(BF16)

    @pl.when(p == 3)
    def _():
        x0pb16 = x0p16_s[rows, :]
        x2pb = 2.0 * _mm(S16_s[rows, :], x1p16_s[...]) - x0pb16.astype(F32)
        hc = (_mm(x0pb16, Wc_ref[0])
              + _mm(x1p16_s[rows, :], Wc_ref[1])
              + _mm(x2pb.astype(BF16), Wc_ref[2])
              + jnp.tile(bc_ref[...], (1, B)))
        c = jnp.tanh(hc)
        u = u16_s[rows, :].astype(F32)
        st = st2_ref[rows, :]
        ov = u * st + (1.0 - u) * c
        for b in range(B):
            out_ref[b, :, :] = ov[:, b * D_H: (b + 1) * D_H]


def _expand_weight(W, d_out):
    W3 = W.T.reshape(D_CAT, M, d_out).transpose(1, 0, 2)
    eye = jnp.eye(B, dtype=W.dtype)
    T = (W3[:, None, :, None, :] * eye[None, :, None, :, None])
    return T.reshape(M, B * D_CAT, B * d_out).astype(BF16)


def kernel(inputs, supports, states, W_ru, b_ru, W_c, b_c):
    n = supports.shape[0]
    in2 = inputs.transpose(1, 0, 2).reshape(n, B * D_IN)
    st2 = states.transpose(1, 0, 2).reshape(n, B * D_H)
    Wru_cat = _expand_weight(W_ru, 2 * D_H)
    Wc_cat = _expand_weight(W_c, D_H)

    out = pl.pallas_call(
        _gru_body,
        grid=(4, NB),
        in_specs=[
            pl.BlockSpec((BN, N), lambda p, i: ((p == 0) * i, 0)),
            pl.BlockSpec((n, B * D_IN), lambda p, i: (0, 0)),
            pl.BlockSpec((n, B * D_H), lambda p, i: (0, 0)),
            pl.BlockSpec((M, DXB, 2 * DH4), lambda p, i: (0, 0, 0)),
            pl.BlockSpec((1, 2 * D_H), lambda p, i: (0, 0)),
            pl.BlockSpec((M, DXB, DH4), lambda p, i: (0, 0, 0)),
            pl.BlockSpec((1, D_H), lambda p, i: (0, 0)),
        ],
        out_specs=pl.BlockSpec((B, BN, D_H), lambda p, i: (0, i, 0)),
        out_shape=jax.ShapeDtypeStruct((B, n, D_H), F32),
        scratch_shapes=[
            pltpu.VMEM((N, N), BF16),
            pltpu.VMEM((N, DXB), BF16),
            pltpu.VMEM((N, DXB), BF16),
            pltpu.VMEM((N, DXB), BF16),
            pltpu.VMEM((N, DXB), BF16),
            pltpu.VMEM((N, DH4), BF16),
        ],
        compiler_params=pltpu.CompilerParams(
            dimension_semantics=("arbitrary", "arbitrary")),
    )(supports, in2, st2, Wru_cat, b_ru.reshape(1, 2 * D_H),
      Wc_cat, b_c.reshape(1, D_H))

    return (out, out)


# speedup vs baseline: 1.8590x; 1.3970x over previous
import jax
import jax.numpy as jnp
from jax.experimental import pallas as pl
from jax.experimental.pallas import tpu as pltpu

N = 4096
BN0 = 128
NB0 = N // BN0
BNC = 512
NBC = N // BNC
B = 4
D_IN = 16
D_H = 32
D_CAT = D_IN + D_H
DXB = D_CAT * B
DH4 = D_H * B
M = 3
F32 = jnp.float32
BF16 = jnp.bfloat16

S0 = NB0
S1 = S0 + NBC
S2 = S1 + NBC
STEPS = S2 + NBC


def _mm(a, b):
    return jax.lax.dot(a, b, preferred_element_type=F32)


def _x0_rows(in2, st2):
    return jnp.concatenate(
        [jnp.concatenate([in2[:, b * D_IN:(b + 1) * D_IN],
                          st2[:, b * D_H:(b + 1) * D_H]], axis=1)
         for b in range(B)], axis=1)


def _gru_body(S_ref, in2_ref, st2_ref, Wru_ref, bru_ref, Wc_ref, bc_ref,
              out_ref, S16_s, x016_s, x116_s, x0p16_s, x1p16_s, u16_s):
    s = pl.program_id(0)

    @pl.when(s == 0)
    def _():
        x016_s[...] = _x0_rows(in2_ref[...].astype(F32),
                               st2_ref[...]).astype(BF16)

    @pl.when(s < S0)
    def _():
        rows = pl.ds(s * BN0, BN0)
        Sb16 = S_ref[...].astype(BF16)
        S16_s[rows, :] = Sb16
        x116_s[rows, :] = _mm(Sb16, x016_s[...]).astype(BF16)

    @pl.when((s >= S0) & (s < S1))
    def _():
        rows = pl.ds((s - S0) * BNC, BNC)
        x0b = _x0_rows(in2_ref[rows, :].astype(F32), st2_ref[rows, :])
        x2b = 2.0 * _mm(S16_s[rows, :], x116_s[...]) - x0b
        h = (_mm(x016_s[rows, :], Wru_ref[0])
             + _mm(x116_s[rows, :], Wru_ref[1])
             + _mm(x2b.astype(BF16), Wru_ref[2])
             + jnp.tile(bru_ref[...], (1, B)))
        ru = jax.nn.sigmoid(h)
        r = jnp.concatenate(
            [ru[:, b * 2 * D_H: b * 2 * D_H + D_H] for b in range(B)],
            axis=1)
        u16_s[rows, :] = jnp.concatenate(
            [ru[:, b * 2 * D_H + D_H: (b + 1) * 2 * D_H] for b in range(B)],
            axis=1).astype(BF16)
        rst = r * st2_ref[rows, :]
        x0p = jnp.concatenate(
            [jnp.concatenate([in2_ref[rows, b * D_IN:(b + 1) * D_IN]
                              .astype(F32),
                              rst[:, b * D_H:(b + 1) * D_H]], axis=1)
             for b in range(B)], axis=1)
        x0p16_s[rows, :] = x0p.astype(BF16)

    @pl.when((s >= S1) & (s < S2))
    def _():
        rows = pl.ds((s - S1) * BNC, BNC)
        x1p16_s[rows, :] = _mm(S16_s[rows, :], x0p16_s[...]).astype(BF16)

    @pl.when(s >= S2)
    def _():
        rows = pl.ds((s - S2) * BNC, BNC)
        x0pb16 = x0p16_s[rows, :]
        x2pb = 2.0 * _mm(S16_s[rows, :], x1p16_s[...]) - x0pb16.astype(F32)
        hc = (_mm(x0pb16, Wc_ref[0])
              + _mm(x1p16_s[rows, :], Wc_ref[1])
              + _mm(x2pb.astype(BF16), Wc_ref[2])
              + jnp.tile(bc_ref[...], (1, B)))
        c = jnp.tanh(hc)
        u = u16_s[rows, :].astype(F32)
        st = st2_ref[rows, :]
        ov = u * st + (1.0 - u) * c
        for b in range(B):
            out_ref[b, :, :] = ov[:, b * D_H: (b + 1) * D_H]


def _expand_weight(W, d_out):
    W3 = W.T.reshape(D_CAT, M, d_out).transpose(1, 0, 2)
    eye = jnp.eye(B, dtype=W.dtype)
    T = (W3[:, None, :, None, :] * eye[None, :, None, :, None])
    return T.reshape(M, B * D_CAT, B * d_out).astype(BF16)


def kernel(inputs, supports, states, W_ru, b_ru, W_c, b_c):
    n = supports.shape[0]
    in2 = inputs.transpose(1, 0, 2).reshape(n, B * D_IN).astype(BF16)
    st2 = states.transpose(1, 0, 2).reshape(n, B * D_H)
    Wru_cat = _expand_weight(W_ru, 2 * D_H)
    Wc_cat = _expand_weight(W_c, D_H)

    out = pl.pallas_call(
        _gru_body,
        grid=(STEPS,),
        in_specs=[
            pl.BlockSpec((BN0, N), lambda s: (s * (s < S0), 0)),
            pl.BlockSpec((n, B * D_IN), lambda s: (0, 0)),
            pl.BlockSpec((n, B * D_H), lambda s: (0, 0)),
            pl.BlockSpec((M, DXB, 2 * DH4), lambda s: (0, 0, 0)),
            pl.BlockSpec((1, 2 * D_H), lambda s: (0, 0)),
            pl.BlockSpec((M, DXB, DH4), lambda s: (0, 0, 0)),
            pl.BlockSpec((1, D_H), lambda s: (0, 0)),
        ],
        out_specs=pl.BlockSpec(
            (B, BNC, D_H), lambda s: (0, (s - S2) * (s >= S2), 0)),
        out_shape=jax.ShapeDtypeStruct((B, n, D_H), F32),
        scratch_shapes=[
            pltpu.VMEM((N, N), BF16),
            pltpu.VMEM((N, DXB), BF16),
            pltpu.VMEM((N, DXB), BF16),
            pltpu.VMEM((N, DXB), BF16),
            pltpu.VMEM((N, DXB), BF16),
            pltpu.VMEM((N, DH4), BF16),
        ],
        compiler_params=pltpu.CompilerParams(
            dimension_semantics=("arbitrary",)),
    )(supports, in2, st2, Wru_cat, b_ru.reshape(1, 2 * D_H),
      Wc_cat, b_c.reshape(1, D_H))

    return (out, out)


# speedup vs baseline: 1.9486x; 1.0482x over previous
import jax
import jax.numpy as jnp
from jax.experimental import pallas as pl
from jax.experimental.pallas import tpu as pltpu

N = 4096
BN0 = 128
NB0 = N // BN0
BNC = 1024
NBC = N // BNC
B = 4
D_IN = 16
D_H = 32
D_CAT = D_IN + D_H
DXB = D_CAT * B
DH4 = D_H * B
M = 3
F32 = jnp.float32
BF16 = jnp.bfloat16

S0 = NB0
S1 = S0 + NBC
S2 = S1 + NBC
STEPS = S2 + NBC


def _mm(a, b):
    return jax.lax.dot(a, b, preferred_element_type=F32)


def _x0_rows(in2, st2):
    return jnp.concatenate(
        [jnp.concatenate([in2[:, b * D_IN:(b + 1) * D_IN],
                          st2[:, b * D_H:(b + 1) * D_H]], axis=1)
         for b in range(B)], axis=1)


def _gru_body(S_ref, in2_ref, st2_ref, Wru_ref, bru_ref, Wc_ref, bc_ref,
              out_ref, S16_s, x016_s, x116_s, x0p16_s, u16_s):
    s = pl.program_id(0)

    @pl.when(s == 0)
    def _():
        x016_s[...] = _x0_rows(in2_ref[...].astype(F32),
                               st2_ref[...]).astype(BF16)

    @pl.when(s < S0)
    def _():
        rows = pl.ds(s * BN0, BN0)
        Sb16 = S_ref[...].astype(BF16)
        S16_s[rows, :] = Sb16
        x116_s[rows, :] = _mm(Sb16, x016_s[...]).astype(BF16)

    @pl.when((s >= S0) & (s < S1))
    def _():
        rows = pl.ds((s - S0) * BNC, BNC)
        x2b = (2.0 * _mm(S16_s[rows, :], x116_s[...])
               - x016_s[rows, :].astype(F32))
        h = (_mm(x016_s[rows, :], Wru_ref[0])
             + _mm(x116_s[rows, :], Wru_ref[1])
             + _mm(x2b.astype(BF16), Wru_ref[2])
             + jnp.tile(bru_ref[...], (1, B)))
        ru = jax.nn.sigmoid(h)
        r = jnp.concatenate(
            [ru[:, b * 2 * D_H: b * 2 * D_H + D_H] for b in range(B)],
            axis=1)
        u16_s[rows, :] = jnp.concatenate(
            [ru[:, b * 2 * D_H + D_H: (b + 1) * 2 * D_H] for b in range(B)],
            axis=1).astype(BF16)
        rst16 = (r * st2_ref[rows, :]).astype(BF16)
        x0p16_s[rows, :] = jnp.concatenate(
            [jnp.concatenate(
                [x016_s[rows, b * D_CAT: b * D_CAT + D_IN],
                 rst16[:, b * D_H:(b + 1) * D_H]], axis=1)
             for b in range(B)], axis=1)

    @pl.when((s >= S1) & (s < S2))
    def _():
        rows = pl.ds((s - S1) * BNC, BNC)
        x116_s[rows, :] = _mm(S16_s[rows, :], x0p16_s[...]).astype(BF16)

    @pl.when(s >= S2)
    def _():
        rows = pl.ds((s - S2) * BNC, BNC)
        x0pb16 = x0p16_s[rows, :]
        x2pb = 2.0 * _mm(S16_s[rows, :], x116_s[...]) - x0pb16.astype(F32)
        hc = (_mm(x0pb16, Wc_ref[0])
              + _mm(x116_s[rows, :], Wc_ref[1])
              + _mm(x2pb.astype(BF16), Wc_ref[2])
              + jnp.tile(bc_ref[...], (1, B)))
        c = jnp.tanh(hc)
        u = u16_s[rows, :].astype(F32)
        st = st2_ref[rows, :]
        ov = u * st + (1.0 - u) * c
        for b in range(B):
            out_ref[b, :, :] = ov[:, b * D_H: (b + 1) * D_H]


def _expand_weight(W, d_out):
    W3 = W.T.reshape(D_CAT, M, d_out).transpose(1, 0, 2)
    eye = jnp.eye(B, dtype=W.dtype)
    T = (W3[:, None, :, None, :] * eye[None, :, None, :, None])
    return T.reshape(M, B * D_CAT, B * d_out).astype(BF16)


def kernel(inputs, supports, states, W_ru, b_ru, W_c, b_c):
    n = supports.shape[0]
    in2 = inputs.transpose(1, 0, 2).reshape(n, B * D_IN).astype(BF16)
    st2 = states.transpose(1, 0, 2).reshape(n, B * D_H)
    Wru_cat = _expand_weight(W_ru, 2 * D_H)
    Wc_cat = _expand_weight(W_c, D_H)

    out = pl.pallas_call(
        _gru_body,
        grid=(STEPS,),
        in_specs=[
            pl.BlockSpec((BN0, N), lambda s: (s * (s < S0), 0)),
            pl.BlockSpec((n, B * D_IN), lambda s: (0, 0)),
            pl.BlockSpec((n, B * D_H), lambda s: (0, 0)),
            pl.BlockSpec((M, DXB, 2 * DH4), lambda s: (0, 0, 0)),
            pl.BlockSpec((1, 2 * D_H), lambda s: (0, 0)),
            pl.BlockSpec((M, DXB, DH4), lambda s: (0, 0, 0)),
            pl.BlockSpec((1, D_H), lambda s: (0, 0)),
        ],
        out_specs=pl.BlockSpec(
            (B, BNC, D_H), lambda s: (0, (s - S2) * (s >= S2), 0)),
        out_shape=jax.ShapeDtypeStruct((B, n, D_H), F32),
        scratch_shapes=[
            pltpu.VMEM((N, N), BF16),
            pltpu.VMEM((N, DXB), BF16),
            pltpu.VMEM((N, DXB), BF16),
            pltpu.VMEM((N, DXB), BF16),
            pltpu.VMEM((N, DH4), BF16),
        ],
        compiler_params=pltpu.CompilerParams(
            dimension_semantics=("arbitrary",)),
    )(supports, in2, st2, Wru_cat, b_ru.reshape(1, 2 * D_H),
      Wc_cat, b_c.reshape(1, D_H))

    return (out, out)


# speedup vs baseline: 2.1885x; 1.1231x over previous
import jax
import jax.numpy as jnp
from jax.experimental import pallas as pl
from jax.experimental.pallas import tpu as pltpu

N = 4096
BN0 = 256
NB0 = N // BN0
BNC = 1024
NBC = N // BNC
B = 4
D_IN = 16
D_H = 32
D_CAT = D_IN + D_H
DXB = D_CAT * B
DH4 = D_H * B
M = 3
F32 = jnp.float32
BF16 = jnp.bfloat16

S0 = NB0
S1 = S0 + NBC
S2 = S1 + NBC
STEPS = S2 + NBC


def _mm(a, b):
    return jax.lax.dot(a, b, preferred_element_type=F32)


def _gru_body(S_ref, x016_ref, st2_ref, Wru_ref, bru_ref, Wc_ref, bc_ref,
              out_ref, S16_s, x116_s, x0p16_s, u16_s):
    s = pl.program_id(0)

    @pl.when(s == 0)
    def _():
        x0p16_s[...] = x016_ref[...]

    @pl.when(s < S0)
    def _():
        rows = pl.ds(s * BN0, BN0)
        Sb16 = S_ref[...].astype(BF16)
        S16_s[rows, :] = Sb16
        x116_s[rows, :] = _mm(Sb16, x016_ref[...]).astype(BF16)

    @pl.when((s >= S0) & (s < S1))
    def _():
        rows = pl.ds((s - S0) * BNC, BNC)
        x2b = (2.0 * _mm(S16_s[rows, :], x116_s[...])
               - x016_ref[rows, :].astype(F32))
        h = (_mm(x016_ref[rows, :], Wru_ref[0])
             + _mm(x116_s[rows, :], Wru_ref[1])
             + _mm(x2b.astype(BF16), Wru_ref[2])
             + jnp.tile(bru_ref[...], (1, B)))
        ru = jax.nn.sigmoid(h)
        st2r = st2_ref[rows, :]
        for b in range(B):
            u16_s[rows, b * D_H:(b + 1) * D_H] = (
                ru[:, b * 2 * D_H + D_H:(b + 1) * 2 * D_H].astype(BF16))
            x0p16_s[rows, b * D_CAT + D_IN:(b + 1) * D_CAT] = (
                ru[:, b * 2 * D_H: b * 2 * D_H + D_H]
                * st2r[:, b * D_H:(b + 1) * D_H]).astype(BF16)

    @pl.when((s >= S1) & (s < S2))
    def _():
        rows = pl.ds((s - S1) * BNC, BNC)
        x116_s[rows, :] = _mm(S16_s[rows, :], x0p16_s[...]).astype(BF16)

    @pl.when(s >= S2)
    def _():
        rows = pl.ds((s - S2) * BNC, BNC)
        x0pb16 = x0p16_s[rows, :]
        x2pb = 2.0 * _mm(S16_s[rows, :], x116_s[...]) - x0pb16.astype(F32)
        hc = (_mm(x0pb16, Wc_ref[0])
              + _mm(x116_s[rows, :], Wc_ref[1])
              + _mm(x2pb.astype(BF16), Wc_ref[2])
              + jnp.tile(bc_ref[...], (1, B)))
        c = jnp.tanh(hc)
        u = u16_s[rows, :].astype(F32)
        st = st2_ref[rows, :]
        ov = u * st + (1.0 - u) * c
        for b in range(B):
            out_ref[b, :, :] = ov[:, b * D_H:(b + 1) * D_H]


def _expand_weight(W, d_out):
    W3 = W.T.reshape(D_CAT, M, d_out).transpose(1, 0, 2)
    eye = jnp.eye(B, dtype=W.dtype)
    T = (W3[:, None, :, None, :] * eye[None, :, None, :, None])
    return T.reshape(M, B * D_CAT, B * d_out).astype(BF16)


def kernel(inputs, supports, states, W_ru, b_ru, W_c, b_c):
    n = supports.shape[0]
    st2 = states.transpose(1, 0, 2).reshape(n, B * D_H)
    x016 = jnp.concatenate([inputs, states], axis=-1).transpose(
        1, 0, 2).reshape(n, DXB).astype(BF16)
    Wru_cat = _expand_weight(W_ru, 2 * D_H)
    Wc_cat = _expand_weight(W_c, D_H)

    out = pl.pallas_call(
        _gru_body,
        grid=(STEPS,),
        in_specs=[
            pl.BlockSpec((BN0, N), lambda s: (s * (s < S0), 0)),
            pl.BlockSpec((n, DXB), lambda s: (0, 0)),
            pl.BlockSpec((n, B * D_H), lambda s: (0, 0)),
            pl.BlockSpec((M, DXB, 2 * DH4), lambda s: (0, 0, 0)),
            pl.BlockSpec((1, 2 * D_H), lambda s: (0, 0)),
            pl.BlockSpec((M, DXB, DH4), lambda s: (0, 0, 0)),
            pl.BlockSpec((1, D_H), lambda s: (0, 0)),
        ],
        out_specs=pl.BlockSpec(
            (B, BNC, D_H), lambda s: (0, (s - S2) * (s >= S2), 0)),
        out_shape=jax.ShapeDtypeStruct((B, n, D_H), F32),
        scratch_shapes=[
            pltpu.VMEM((N, N), BF16),
            pltpu.VMEM((N, DXB), BF16),
            pltpu.VMEM((N, DXB), BF16),
            pltpu.VMEM((N, DH4), BF16),
        ],
        compiler_params=pltpu.CompilerParams(
            dimension_semantics=("arbitrary",)),
    )(supports, x016, st2, Wru_cat, b_ru.reshape(1, 2 * D_H),
      Wc_cat, b_c.reshape(1, D_H))

    return (out, out)
